# baseline (speedup 1.0000x reference)
import functools

import jax
import jax.numpy as jnp
import numpy as np
from jax import lax
from jax.experimental import pallas as pl
from jax.experimental.pallas import tpu as pltpu
from jax.experimental.pallas import tpu_sc as plsc

_OFFSETS = np.array([0, 6, 12, 15, 25, 32, 40, 42, 46, 53, 53], dtype=np.int64)
_PADDING_IDX = 61
_NUM_BAGS = 10
_DIM = 5
_LANES = 16
_NSLOTS = _NUM_BAGS * _DIM
_NCHUNKS = 4
_INP_PAD = 64
_W_PAD = 512


def _build_tables():
    lens = (_OFFSETS[1:] - _OFFSETS[:-1]).astype(np.int32)
    jmax, pos, dvec = [], [], []
    for r in range(_NCHUNKS):
        slots = np.arange(r * _LANES, (r + 1) * _LANES)
        bags = slots // _DIM
        valid = slots < _NSLOTS
        dv = np.where(valid, slots % _DIM, 0).astype(np.int32)
        jm = int(max([lens[b] for b, v in zip(bags, valid) if v], default=0))
        pr = []
        for j in range(jm):
            p = np.full((_LANES,), _INP_PAD - 1, np.int32)
            for l in range(_LANES):
                if valid[l] and j < lens[bags[l]]:
                    p[l] = int(_OFFSETS[bags[l]]) + j
            pr.append(p)
        jmax.append(jm)
        pos.append(pr)
        dvec.append(dv)
    return jmax, pos, dvec


_JMAX, _POS, _DVEC = _build_tables()

_TAB_ROWS = 32
_POS_ROW = [0] * _NCHUNKS


def _build_tab():
    rows = list(_DVEC)
    for r in range(_NCHUNKS):
        _POS_ROW[r] = len(rows)
        rows.extend(_POS[r])
    assert len(rows) <= _TAB_ROWS
    while len(rows) < _TAB_ROWS:
        rows.append(np.zeros((_LANES,), np.int32))
    return np.stack(rows).astype(np.int32)


_TAB = _build_tab()

_mesh = plsc.VectorSubcoreMesh(core_axis_name="c", subcore_axis_name="s")


@functools.partial(
    pl.kernel,
    out_type=jax.ShapeDtypeStruct((_NCHUNKS * _LANES,), jnp.float32),
    mesh=_mesh,
    scratch_types=[
        pltpu.VMEM((_INP_PAD,), jnp.int32),
        pltpu.VMEM((_W_PAD,), jnp.float32),
        pltpu.VMEM((_TAB_ROWS, _LANES), jnp.int32),
        pltpu.VMEM((_NCHUNKS * _LANES,), jnp.float32),
    ],
    compiler_params=pltpu.CompilerParams(needs_layout_passes=False),
)
def _bag_mean_sc(inp_hbm, w_hbm, tab_hbm, out_hbm, inp_v, w_v, tab_v, out_v):
    c = lax.axis_index("c")
    s = lax.axis_index("s")

    @pl.when(jnp.logical_and(c == 0, s == 0))
    def _():
        pltpu.sync_copy(inp_hbm, inp_v)
        pltpu.sync_copy(w_hbm, w_v)
        pltpu.sync_copy(tab_hbm, tab_v)
        for r in range(_NCHUNKS):
            acc = jnp.zeros((_LANES,), jnp.float32)
            cnt = jnp.zeros((_LANES,), jnp.float32)
            dvec = tab_v[r]
            for j in range(_JMAX[r]):
                posv = tab_v[_POS_ROW[r] + j]
                idx = plsc.load_gather(inp_v, [posv])
                mf = jnp.where(idx != _PADDING_IDX, 1.0, 0.0).astype(jnp.float32)
                idxc = jnp.minimum(jnp.maximum(idx, 0), 99)
                vals = plsc.load_gather(w_v, [idxc * _DIM + dvec])
                acc = acc + vals * mf
                cnt = cnt + mf
            out_v[pl.ds(r * _LANES, _LANES)] = acc / jnp.maximum(cnt, 1.0)
        pltpu.sync_copy(out_v, out_hbm)


def kernel(input, weight):
    inp = jnp.concatenate(
        [
            input.astype(jnp.int32),
            jnp.full((_INP_PAD - input.shape[0],), _PADDING_IDX, jnp.int32),
        ]
    )
    w_flat = jnp.concatenate(
        [
            weight.reshape(-1).astype(jnp.float32),
            jnp.zeros((_W_PAD - weight.size,), jnp.float32),
        ]
    )
    out = _bag_mean_sc(inp, w_flat, jnp.asarray(_TAB))
    return out[:_NSLOTS].reshape(_NUM_BAGS, _DIM)


# speedup vs baseline: 1.0155x; 1.0155x over previous
import functools

import jax
import jax.numpy as jnp
import numpy as np
from jax import lax
from jax.experimental import pallas as pl
from jax.experimental.pallas import tpu as pltpu
from jax.experimental.pallas import tpu_sc as plsc

_OFFSETS = np.array([0, 6, 12, 15, 25, 32, 40, 42, 46, 53, 53], dtype=np.int64)
_PADDING_IDX = 61
_NUM_BAGS = 10
_DIM = 5
_LANES = 16
_N_IDX = 53
_N_ROWS = 100
_NSLOTS = _NUM_BAGS * _DIM
_NCHUNKS = 4
_INVALID_POS = 63


def _build_tables():
    lens = (_OFFSETS[1:] - _OFFSETS[:-1]).astype(np.int32)
    jmax, pos, dvec = [], [], []
    for r in range(_NCHUNKS):
        slots = np.arange(r * _LANES, (r + 1) * _LANES)
        bags = slots // _DIM
        valid = slots < _NSLOTS
        dvec.append(np.where(valid, slots % _DIM, 0).astype(np.int32))
        jm = int(max([lens[b] for b, v in zip(bags, valid) if v], default=0))
        rows = []
        for j in range(jm):
            p = np.full((_LANES,), _INVALID_POS, np.int32)
            for l in range(_LANES):
                if valid[l] and j < lens[bags[l]]:
                    p[l] = int(_OFFSETS[bags[l]]) + j
            rows.append(p)
        jmax.append(jm)
        pos.append(rows)
    return jmax, pos, dvec


_JMAX, _POS, _DVEC = _build_tables()

_POS_ROW = [0] * _NCHUNKS


def _build_tab():
    rows = list(_DVEC)
    for r in range(_NCHUNKS):
        _POS_ROW[r] = len(rows)
        rows.extend(_POS[r])
    while len(rows) % 8:
        rows.append(np.zeros((_LANES,), np.int32))
    return np.stack(rows).astype(np.int32)


_TAB = _build_tab()
_TAB_ROWS = _TAB.shape[0]

_mesh = plsc.VectorSubcoreMesh(core_axis_name="c", subcore_axis_name="s")


@functools.partial(
    pl.kernel,
    out_type=jax.ShapeDtypeStruct((_NSLOTS,), jnp.float32),
    mesh=_mesh,
    scratch_types=[
        pltpu.VMEM((_N_IDX,), jnp.int32),
        pltpu.VMEM((_N_ROWS, _DIM), jnp.float32),
        pltpu.VMEM((_TAB_ROWS, _LANES), jnp.int32),
        pltpu.VMEM((_NCHUNKS * _LANES,), jnp.float32),
        pltpu.SemaphoreType.DMA,
        pltpu.SemaphoreType.DMA,
        pltpu.SemaphoreType.DMA,
    ],
    compiler_params=pltpu.CompilerParams(needs_layout_passes=False),
)
def _bag_mean_sc(inp_hbm, w_hbm, tab_hbm, out_hbm, inp_v, w_v, tab_v, out_v,
                 sem1, sem2, sem3):
    c = lax.axis_index("c")
    s = lax.axis_index("s")

    @pl.when(jnp.logical_and(c == 0, s == 0))
    def _():
        cp1 = pltpu.async_copy(inp_hbm, inp_v, sem1)
        cp2 = pltpu.async_copy(w_hbm, w_v, sem2)
        cp3 = pltpu.async_copy(tab_hbm, tab_v, sem3)
        cp1.wait()
        cp2.wait()
        cp3.wait()
        for r in range(_NCHUNKS):
            acc = jnp.zeros((_LANES,), jnp.float32)
            cnt = jnp.zeros((_LANES,), jnp.float32)
            dvec = tab_v[r]
            for j in range(_JMAX[r]):
                posv = tab_v[_POS_ROW[r] + j]
                lane_on = posv < _N_IDX
                idx = plsc.load_gather(inp_v, [jnp.minimum(posv, _N_IDX - 1)])
                mf = jnp.where(
                    jnp.logical_and(lane_on, idx != _PADDING_IDX), 1.0, 0.0
                ).astype(jnp.float32)
                idxc = jnp.minimum(jnp.maximum(idx, 0), _N_ROWS - 1)
                vals = plsc.load_gather(w_v, [idxc, dvec])
                acc = acc + vals * mf
                cnt = cnt + mf
            out_v[pl.ds(r * _LANES, _LANES)] = acc / jnp.maximum(cnt, 1.0)
        pltpu.sync_copy(out_v.at[pl.ds(0, _NSLOTS)], out_hbm)


def kernel(input, weight):
    out = _bag_mean_sc(input, weight, jnp.asarray(_TAB))
    return out.reshape(_NUM_BAGS, _DIM)


# speedup vs baseline: 1.0834x; 1.0668x over previous
import functools

import jax
import jax.numpy as jnp
import numpy as np
from jax import lax
from jax.experimental import pallas as pl
from jax.experimental.pallas import tpu as pltpu
from jax.experimental.pallas import tpu_sc as plsc

_OFFSETS = np.array([0, 6, 12, 15, 25, 32, 40, 42, 46, 53, 53], dtype=np.int64)
_PADDING_IDX = 61
_NUM_BAGS = 10
_DIM = 5
_LANES = 16
_N_IDX = 53
_N_ROWS = 100
_NSLOTS = _NUM_BAGS * _DIM
_NCHUNKS = 4
_INVALID_POS = 63


def _build_tables():
    lens = (_OFFSETS[1:] - _OFFSETS[:-1]).astype(np.int32)
    jmax, pos, dvec = [], [], []
    for r in range(_NCHUNKS):
        slots = np.arange(r * _LANES, (r + 1) * _LANES)
        bags = slots // _DIM
        valid = slots < _NSLOTS
        dvec.append(np.where(valid, slots % _DIM, 0).astype(np.int32))
        jm = int(max([lens[b] for b, v in zip(bags, valid) if v], default=0))
        rows = []
        for j in range(jm):
            p = np.full((_LANES,), _INVALID_POS, np.int32)
            for l in range(_LANES):
                if valid[l] and j < lens[bags[l]]:
                    p[l] = int(_OFFSETS[bags[l]]) + j
            rows.append(p)
        jmax.append(jm)
        pos.append(rows)
    return jmax, pos, dvec


_JMAX, _POS, _DVEC = _build_tables()

_POS_ROW = [0] * _NCHUNKS


def _build_tab():
    rows = list(_DVEC)
    for r in range(_NCHUNKS):
        _POS_ROW[r] = len(rows)
        rows.extend(_POS[r])
    while len(rows) % 8:
        rows.append(np.zeros((_LANES,), np.int32))
    return np.stack(rows).astype(np.int32)


_TAB = _build_tab()
_TAB_ROWS = _TAB.shape[0]

_mesh = plsc.VectorSubcoreMesh(
    core_axis_name="c", subcore_axis_name="s", num_cores=1, num_subcores=1
)


@functools.partial(
    pl.kernel,
    out_type=jax.ShapeDtypeStruct((_NSLOTS,), jnp.float32),
    mesh=_mesh,
    scratch_types=[
        pltpu.VMEM((_N_IDX,), jnp.int32),
        pltpu.VMEM((_N_ROWS, _DIM), jnp.float32),
        pltpu.VMEM((_TAB_ROWS, _LANES), jnp.int32),
        pltpu.VMEM((_NCHUNKS * _LANES,), jnp.float32),
        pltpu.SemaphoreType.DMA,
        pltpu.SemaphoreType.DMA,
        pltpu.SemaphoreType.DMA,
    ],
    compiler_params=pltpu.CompilerParams(
        needs_layout_passes=False,
        disable_bounds_checks=True,
        disable_semaphore_checks=True,
    ),
)
def _bag_mean_sc(inp_hbm, w_hbm, tab_hbm, out_hbm, inp_v, w_v, tab_v, out_v,
                 sem1, sem2, sem3):
    cp1 = pltpu.async_copy(inp_hbm, inp_v, sem1)
    cp2 = pltpu.async_copy(w_hbm, w_v, sem2)
    cp3 = pltpu.async_copy(tab_hbm, tab_v, sem3)
    cp1.wait()
    cp2.wait()
    cp3.wait()
    for r in range(_NCHUNKS):
        acc = jnp.zeros((_LANES,), jnp.float32)
        cnt = jnp.zeros((_LANES,), jnp.float32)
        dvec = tab_v[r]
        for j in range(_JMAX[r]):
            posv = tab_v[_POS_ROW[r] + j]
            lane_on = posv < _N_IDX
            idx = plsc.load_gather(inp_v, [jnp.minimum(posv, _N_IDX - 1)])
            mf = jnp.where(
                jnp.logical_and(lane_on, idx != _PADDING_IDX), 1.0, 0.0
            ).astype(jnp.float32)
            idxc = jnp.minimum(jnp.maximum(idx, 0), _N_ROWS - 1)
            vals = plsc.load_gather(w_v, [idxc, dvec])
            acc = acc + vals * mf
            cnt = cnt + mf
        out_v[pl.ds(r * _LANES, _LANES)] = acc / jnp.maximum(cnt, 1.0)
    pltpu.sync_copy(out_v.at[pl.ds(0, _NSLOTS)], out_hbm)


def kernel(input, weight):
    out = _bag_mean_sc(input, weight, jnp.asarray(_TAB))
    return out.reshape(_NUM_BAGS, _DIM)
